# Initial kernel scaffold; baseline (speedup 1.0000x reference)
import jax
import jax.numpy as jnp
from jax.experimental import pallas as pl


def kernel(x, emb_table, W, b):
    raise NotImplementedError("write your pallas kernel here")


# speedup vs baseline: 1.5275x; 1.5275x over previous
import functools

import jax
import jax.numpy as jnp
from jax import lax
from jax.experimental import pallas as pl
from jax.experimental.pallas import tpu as pltpu
from jax.experimental.pallas import tpu_sc as plsc

_VOCAB = 1000000
_EMBED = 64
_OUT = 3
_DPAD = 16
_BATCH = 16384
_HIST = 50
_NIDX = _BATCH * _HIST

_NC, _NS = 2, 16
_NW = _NC * _NS
_BPW = _NIDX // _NW
_CHUNK = 3200
_NCHUNK = _BPW // _CHUNK

_MM_ROWS = 8000


def _mm_body(t_ref, w_ref, b_ref, o_ref):
    o_ref[...] = (
        jnp.dot(t_ref[...], w_ref[...], preferred_element_type=jnp.float32)
        + b_ref[...]
    )


def _project_table(emb_table, w_pad, b_pad):
    grid = _VOCAB // _MM_ROWS
    return pl.pallas_call(
        _mm_body,
        grid=(grid,),
        in_specs=[
            pl.BlockSpec((_MM_ROWS, _EMBED), lambda i: (i, 0)),
            pl.BlockSpec((_EMBED, _DPAD), lambda i: (0, 0)),
            pl.BlockSpec((1, _DPAD), lambda i: (0, 0)),
        ],
        out_specs=pl.BlockSpec((_MM_ROWS, _DPAD), lambda i: (i, 0)),
        out_shape=jax.ShapeDtypeStruct((_VOCAB, _DPAD), jnp.float32),
    )(emb_table, w_pad, b_pad)


_sc_mesh = plsc.VectorSubcoreMesh(core_axis_name="c", subcore_axis_name="s")


@functools.partial(
    pl.kernel,
    mesh=_sc_mesh,
    compiler_params=pltpu.CompilerParams(use_tc_tiling_on_sc=False),
    out_type=jax.ShapeDtypeStruct((_NIDX, _DPAD), jnp.float32),
    scratch_types=[
        pltpu.VMEM((_CHUNK,), jnp.int32),
        pltpu.VMEM((_CHUNK, _DPAD), jnp.float32),
        pltpu.SemaphoreType.DMA,
    ],
)
def _gather_sc(proj_hbm, idx_hbm, out_hbm, idx_v, rows_v, sem):
    wid = lax.axis_index("s") * _NC + lax.axis_index("c")
    base = wid * _BPW
    for ci in range(_NCHUNK):
        off = base + ci * _CHUNK
        pltpu.sync_copy(idx_hbm.at[pl.ds(off, _CHUNK)], idx_v)
        pltpu.async_copy(proj_hbm.at[idx_v], rows_v, sem).wait()
        pltpu.sync_copy(rows_v, out_hbm.at[pl.ds(off, _CHUNK)])


def kernel(x, emb_table, W, b):
    w_pad = jnp.zeros((_EMBED, _DPAD), jnp.float32).at[:, :_OUT].set(W)
    b_pad = jnp.zeros((1, _DPAD), jnp.float32).at[0, :_OUT].set(b)
    proj = _project_table(emb_table, w_pad, b_pad)
    rows = _gather_sc(proj, x.reshape(-1))
    return rows[:, :_OUT].reshape(_BATCH, _HIST, _OUT)
